# baseline (speedup 1.0000x reference)
import jax
import jax.numpy as jnp
from jax import lax
from jax.experimental import pallas as pl
from jax.experimental.pallas import tpu as pltpu
from jax.experimental.pallas import tpu_sc as plsc

_N = 16384
_T = 16
_BT = _N // _T
_VR = _BT // 16
_REG = 0.1


def _sdiv(a, b):
  av = jnp.broadcast_to(a, (16,)).astype(jnp.float32)
  bv = jnp.broadcast_to(b, (16,)).astype(jnp.float32)
  return jnp.max(av / bv)


def _scalar_read(ref, i):
  idx = jnp.broadcast_to(jnp.int32(i), (16,))
  return jnp.max(plsc.load_gather(ref, [idx]))


def _scalar_write(ref, i, val):
  lane = lax.iota(jnp.int32, 16)
  idx = jnp.broadcast_to(jnp.int32(i), (16,))
  v = jnp.broadcast_to(val, (16,))
  plsc.store_scatter(ref, [idx], v, mask=lane == 0)


def _sc_body(x_hbm, iseq_hbm, out_hbm, A, B, Iv, nbr, pbuf, pball, obuf,
             sp_a, sp_b, sp_part, s_full, i_full, sums, cnts, isem):
  cid = lax.axis_index("c")
  sid = lax.axis_index("s")
  base = sid * _BT
  lane = lax.iota(jnp.int32, 16)

  icopy = pltpu.async_copy(iseq_hbm.at[pl.ds(base, _BT)], Iv, isem)
  pltpu.sync_copy(x_hbm.at[pl.ds(base, _BT)], A)

  def vsort_pass(size):
    if size >= _N:
      @plsc.parallel_loop(0, _VR, unroll=8)
      def _(v):
        off = v * 16
        A[pl.ds(off, 16)] = jnp.sort(A[pl.ds(off, 16)])
    else:
      @plsc.parallel_loop(0, _VR, unroll=8)
      def _(v):
        off = v * 16
        vec = A[pl.ds(off, 16)]
        srt = jnp.sort(vec)
        rev = srt[::-1]
        asc = ((base + off) & size) == 0
        A[pl.ds(off, 16)] = jnp.where(asc, srt, rev)

  def pair_substage(size, j):
    dv = 1 << (j - 4)

    @plsc.parallel_loop(0, _VR // 2, unroll=8)
    def _(u):
      v = ((u >> (j - 4)) << (j - 3)) | (u & (dv - 1))
      p = v + dv
      a = A[pl.ds(v * 16, 16)]
      b = A[pl.ds(p * 16, 16)]
      asc = ((base + v * 16) & size) == 0
      lo = jnp.minimum(a, b)
      hi = jnp.maximum(a, b)
      A[pl.ds(v * 16, 16)] = jnp.where(asc, lo, hi)
      A[pl.ds(p * 16, 16)] = jnp.where(asc, hi, lo)

  spbufs = [sp_a, sp_b]
  npub = [0]

  def publish():
    sp = spbufs[npub[0] % 2]
    npub[0] += 1
    pltpu.sync_copy(A, sp.at[pl.ds(base, _BT)])
    plsc.subcore_barrier()
    return sp

  def cross_substage(size, j):
    dt = 1 << (j - 10)
    sp = publish()
    partner = (sid ^ dt) * _BT
    pltpu.sync_copy(sp.at[pl.ds(partner, _BT)], B)
    keep_lo = ((sid & dt) == 0) == ((base & size) == 0)

    @plsc.parallel_loop(0, _VR, unroll=8)
    def _(v):
      a = A[pl.ds(v * 16, 16)]
      b = B[pl.ds(v * 16, 16)]
      lo = jnp.minimum(a, b)
      hi = jnp.maximum(a, b)
      A[pl.ds(v * 16, 16)] = jnp.where(keep_lo, lo, hi)

  vsort_pass(16)
  size = 32
  while size <= _N:
    j = size.bit_length() - 2
    while j >= 4:
      if j >= 10:
        cross_substage(size, j)
      else:
        pair_substage(size, j)
      j -= 1
    vsort_pass(size)
    size *= 2

  sp_fin = publish()
  nb_start = pl.multiple_of(jnp.maximum(base - 16, 0), 16)
  pltpu.sync_copy(sp_fin.at[pl.ds(nb_start, 16)], nbr)
  bnd = nbr[...][15]
  icopy.wait()

  zero = jnp.zeros((16,), jnp.float32)

  @plsc.parallel_loop(0, _VR, unroll=8, carry=(zero, zero))
  def scan_carry(v, carry):
    runmax, violmax = carry
    loc = v * 16 + lane
    cur = A[pl.ds(v * 16, 16)]
    isq = Iv[pl.ds(v * 16, 16)]
    gm = (base + loc).astype(jnp.float32)
    z_cur = (gm + 1.0) / _REG
    cur_eff = z_cur - (z_cur - cur)
    runmax = jnp.maximum(runmax, jnp.abs(isq - cur_eff))
    prev = plsc.load_gather(A, [jnp.maximum(loc - 1, 0)])
    prev = jnp.where(loc == 0, bnd, prev)
    viol = ((gm / _REG - prev) > (z_cur - cur)) & (base + loc >= 1)
    violmax = jnp.maximum(violmax, jnp.where(viol, 1.0, 0.0))
    return runmax, violmax

  runmax, violmax = scan_carry
  pvec = jnp.where(lane == 0, jnp.max(runmax),
                   jnp.where(lane == 1, jnp.max(violmax), 0.0))
  pbuf[...] = pvec
  pltpu.sync_copy(pbuf, sp_part.at[pl.ds(sid * 16, 16)])
  plsc.subcore_barrier()

  @pl.when(sid == 0)
  def _final():
    pltpu.sync_copy(sp_part, pball)
    def red(t, acc):
      return jnp.maximum(acc, pball[pl.ds(t * 16, 16)])
    acc = lax.fori_loop(0, _T, red, jnp.full((16,), -jnp.inf, jnp.float32))
    fastmax = jnp.max(jnp.where(lane == 0, acc, -jnp.inf))
    viol_any = jnp.max(jnp.where(lane == 1, acc, 0.0)) > 0.5

    def fallback():
      pltpu.sync_copy(sp_fin, s_full)
      pltpu.sync_copy(iseq_hbm, i_full)

      def y_at(i):
        z = _sdiv((_N - i).astype(jnp.float32), _REG)
        return z - _scalar_read(s_full, _N - 1 - i)

      def push(i, p):
        yi = y_at(i)

        def mcond(st):
          cs, cc, q = st
          qm = jnp.maximum(q - 1, 0)
          return (q > 0) & (
              _sdiv(cs, cc) > _sdiv(_scalar_read(sums, qm),
                                    _scalar_read(cnts, qm)))

        def mbody(st):
          cs, cc, q = st
          return (cs + _scalar_read(sums, q - 1),
                  cc + _scalar_read(cnts, q - 1), q - 1)

        cs, cc, q = lax.while_loop(
            mcond, mbody, (yi, jnp.float32(1.0), p))
        _scalar_write(sums, q, cs)
        _scalar_write(cnts, q, cc)
        return q + 1

      p = lax.fori_loop(0, _N, push, jnp.int32(0))

      def blk_cond(st):
        q, start, _ = st
        return q < p

      def blk_body(st):
        q, start, rmax = st
        mean = _sdiv(_scalar_read(sums, q), _scalar_read(cnts, q))
        cnt = _scalar_read(cnts, q).astype(jnp.int32)

        def elem(i, rm):
          z = _sdiv((_N - i).astype(jnp.float32), _REG)
          xd = z - mean
          tgt = _scalar_read(i_full, _N - 1 - i)
          return jnp.maximum(rm, jnp.abs(tgt - xd))

        rmax = lax.fori_loop(start, start + cnt, elem, rmax)
        return q + 1, start + cnt, rmax

      _, _, rmax = lax.while_loop(
          blk_cond, blk_body,
          (jnp.int32(0), jnp.int32(0), jnp.float32(-jnp.inf)))
      return rmax

    res = lax.cond(viol_any, fallback, lambda: fastmax)
    obuf[...] = jnp.broadcast_to(res, (16,))

    @pl.when(cid == 0)
    def _write():
      pltpu.sync_copy(obuf, out_hbm)


@jax.jit
def kernel(x, i_seq):
  mesh = plsc.VectorSubcoreMesh(core_axis_name="c", subcore_axis_name="s")
  f = pl.kernel(
      _sc_body,
      out_type=jax.ShapeDtypeStruct((16,), jnp.float32),
      mesh=mesh,
      compiler_params=pltpu.CompilerParams(needs_layout_passes=False),
      scratch_types=[
          pltpu.VMEM((_BT,), jnp.float32),
          pltpu.VMEM((_BT,), jnp.float32),
          pltpu.VMEM((_BT,), jnp.float32),
          pltpu.VMEM((16,), jnp.float32),
          pltpu.VMEM((16,), jnp.float32),
          pltpu.VMEM((_T * 16,), jnp.float32),
          pltpu.VMEM((16,), jnp.float32),
          pltpu.VMEM_SHARED((_N,), jnp.float32),
          pltpu.VMEM_SHARED((_N,), jnp.float32),
          pltpu.VMEM_SHARED((_T * 16,), jnp.float32),
          pltpu.VMEM((_N,), jnp.float32),
          pltpu.VMEM((_N,), jnp.float32),
          pltpu.VMEM((_N,), jnp.float32),
          pltpu.VMEM((_N,), jnp.float32),
          pltpu.SemaphoreType.DMA,
      ],
  )
  out = f(x, i_seq)
  return out[0].reshape(())


# speedup vs baseline: 1.0558x; 1.0558x over previous
import jax
import jax.numpy as jnp
from jax import lax
from jax.experimental import pallas as pl
from jax.experimental.pallas import tpu as pltpu
from jax.experimental.pallas import tpu_sc as plsc

_N = 16384
_T = 16
_BT = _N // _T
_VR = _BT // 16
_REG = 0.1


def _sdiv(a, b):
  av = jnp.broadcast_to(a, (16,)).astype(jnp.float32)
  bv = jnp.broadcast_to(b, (16,)).astype(jnp.float32)
  return jnp.max(av / bv)


def _scalar_read(ref, i):
  idx = jnp.broadcast_to(jnp.int32(i), (16,))
  return jnp.max(plsc.load_gather(ref, [idx]))


def _scalar_write(ref, i, val):
  lane = lax.iota(jnp.int32, 16)
  idx = jnp.broadcast_to(jnp.int32(i), (16,))
  v = jnp.broadcast_to(val, (16,))
  plsc.store_scatter(ref, [idx], v, mask=lane == 0)


def _sc_body(x_hbm, iseq_hbm, out_hbm, A, B, Iv, nbr, pbuf, pball, obuf,
             sp_a, sp_b, sp_part, s_full, i_full, sums, cnts, isem):
  cid = lax.axis_index("c")
  sid = lax.axis_index("s")
  base = sid * _BT
  lane = lax.iota(jnp.int32, 16)

  icopy = pltpu.async_copy(iseq_hbm.at[pl.ds(base, _BT)], Iv, isem)
  pltpu.sync_copy(x_hbm.at[pl.ds(base, _BT)], A)

  def vsort_pass(size):
    if size >= _N:
      @plsc.parallel_loop(0, _VR, unroll=8)
      def _(v):
        off = v * 16
        A[pl.ds(off, 16)] = jnp.sort(A[pl.ds(off, 16)])
    else:
      @plsc.parallel_loop(0, _VR, unroll=8)
      def _(v):
        off = v * 16
        vec = A[pl.ds(off, 16)]
        srt = jnp.sort(vec)
        rev = srt[::-1]
        asc = ((base + off) & size) == 0
        A[pl.ds(off, 16)] = jnp.where(asc, srt, rev)

  def pair_substage(size, j):
    dv = 1 << (j - 4)

    @plsc.parallel_loop(0, _VR // 2, unroll=8)
    def _(u):
      v = ((u >> (j - 4)) << (j - 3)) | (u & (dv - 1))
      p = v + dv
      a = A[pl.ds(v * 16, 16)]
      b = A[pl.ds(p * 16, 16)]
      asc = ((base + v * 16) & size) == 0
      lo = jnp.minimum(a, b)
      hi = jnp.maximum(a, b)
      A[pl.ds(v * 16, 16)] = jnp.where(asc, lo, hi)
      A[pl.ds(p * 16, 16)] = jnp.where(asc, hi, lo)

  spbufs = [sp_a, sp_b]
  npub = [0]

  def publish():
    sp = spbufs[npub[0] % 2]
    npub[0] += 1
    pltpu.sync_copy(A, sp.at[pl.ds(base, _BT)])
    plsc.subcore_barrier()
    return sp

  def cross_substage(size, j):
    dt = 1 << (j - 10)
    sp = publish()
    partner = (sid ^ dt) * _BT
    pltpu.sync_copy(sp.at[pl.ds(partner, _BT)], B)
    keep_lo = ((sid & dt) == 0) == ((base & size) == 0)

    @plsc.parallel_loop(0, _VR, unroll=8)
    def _(v):
      a = A[pl.ds(v * 16, 16)]
      b = B[pl.ds(v * 16, 16)]
      lo = jnp.minimum(a, b)
      hi = jnp.maximum(a, b)
      A[pl.ds(v * 16, 16)] = jnp.where(keep_lo, lo, hi)

  vsort_pass(16)
  size = 32
  while size <= _N:
    j = size.bit_length() - 2
    while j >= 4:
      if j >= 10:
        cross_substage(size, j)
      else:
        pair_substage(size, j)
      j -= 1
    vsort_pass(size)
    size *= 2

  sp_fin = publish()
  nb_start = pl.multiple_of(jnp.maximum(base - 16, 0), 16)
  pltpu.sync_copy(sp_fin.at[pl.ds(nb_start, 16)], nbr)
  bnd = nbr[...][15]
  icopy.wait()

  zero = jnp.zeros((16,), jnp.float32)

  @plsc.parallel_loop(0, _VR, unroll=8, carry=(zero, zero))
  def scan_carry(v, carry):
    runmax, violmax = carry
    loc = v * 16 + lane
    cur = A[pl.ds(v * 16, 16)]
    isq = Iv[pl.ds(v * 16, 16)]
    gm = (base + loc).astype(jnp.float32)
    z_cur = (gm + 1.0) / _REG
    cur_eff = z_cur - (z_cur - cur)
    runmax = jnp.maximum(runmax, jnp.abs(isq - cur_eff))
    prev = plsc.load_gather(A, [jnp.maximum(loc - 1, 0)])
    prev = jnp.where(loc == 0, bnd, prev)
    viol = ((gm / _REG - prev) > (z_cur - cur)) & (base + loc >= 1)
    violmax = jnp.maximum(violmax, jnp.where(viol, 1.0, 0.0))
    return runmax, violmax

  runmax, violmax = scan_carry
  pvec = jnp.where(lane == 0, jnp.max(runmax),
                   jnp.where(lane == 1, jnp.max(violmax), 0.0))
  pbuf[...] = pvec
  pltpu.sync_copy(pbuf, sp_part.at[pl.ds(sid * 16, 16)])
  plsc.subcore_barrier()

  @pl.when(sid == 0)
  def _final():
    pltpu.sync_copy(sp_part, pball)
    def red(t, acc):
      return jnp.maximum(acc, pball[pl.ds(t * 16, 16)])
    acc = lax.fori_loop(0, _T, red, jnp.full((16,), -jnp.inf, jnp.float32))
    fastmax = jnp.max(jnp.where(lane == 0, acc, -jnp.inf))
    viol_any = jnp.max(jnp.where(lane == 1, acc, 0.0)) > 0.5

    def fallback():
      pltpu.sync_copy(sp_fin, s_full)
      pltpu.sync_copy(iseq_hbm, i_full)

      def y_at(i):
        z = _sdiv((_N - i).astype(jnp.float32), _REG)
        return z - _scalar_read(s_full, _N - 1 - i)

      def push(i, p):
        yi = y_at(i)

        def mcond(st):
          cs, cc, q = st
          qm = jnp.maximum(q - 1, 0)
          return (q > 0) & (
              _sdiv(cs, cc) > _sdiv(_scalar_read(sums, qm),
                                    _scalar_read(cnts, qm)))

        def mbody(st):
          cs, cc, q = st
          return (cs + _scalar_read(sums, q - 1),
                  cc + _scalar_read(cnts, q - 1), q - 1)

        cs, cc, q = lax.while_loop(
            mcond, mbody, (yi, jnp.float32(1.0), p))
        _scalar_write(sums, q, cs)
        _scalar_write(cnts, q, cc)
        return q + 1

      p = lax.fori_loop(0, _N, push, jnp.int32(0))

      def blk_cond(st):
        q, start, _ = st
        return q < p

      def blk_body(st):
        q, start, rmax = st
        mean = _sdiv(_scalar_read(sums, q), _scalar_read(cnts, q))
        cnt = _scalar_read(cnts, q).astype(jnp.int32)

        def elem(i, rm):
          z = _sdiv((_N - i).astype(jnp.float32), _REG)
          xd = z - mean
          tgt = _scalar_read(i_full, _N - 1 - i)
          return jnp.maximum(rm, jnp.abs(tgt - xd))

        rmax = lax.fori_loop(start, start + cnt, elem, rmax)
        return q + 1, start + cnt, rmax

      _, _, rmax = lax.while_loop(
          blk_cond, blk_body,
          (jnp.int32(0), jnp.int32(0), jnp.float32(-jnp.inf)))
      return rmax

    res = lax.cond(viol_any, fallback, lambda: fastmax)
    obuf[...] = jnp.broadcast_to(res, (16,))

    @pl.when(cid == 0)
    def _write():
      pltpu.sync_copy(obuf, out_hbm)


@jax.jit
def kernel(x, i_seq):
  mesh = plsc.VectorSubcoreMesh(core_axis_name="c", subcore_axis_name="s", num_cores=1)
  f = pl.kernel(
      _sc_body,
      out_type=jax.ShapeDtypeStruct((16,), jnp.float32),
      mesh=mesh,
      compiler_params=pltpu.CompilerParams(needs_layout_passes=False),
      scratch_types=[
          pltpu.VMEM((_BT,), jnp.float32),
          pltpu.VMEM((_BT,), jnp.float32),
          pltpu.VMEM((_BT,), jnp.float32),
          pltpu.VMEM((16,), jnp.float32),
          pltpu.VMEM((16,), jnp.float32),
          pltpu.VMEM((_T * 16,), jnp.float32),
          pltpu.VMEM((16,), jnp.float32),
          pltpu.VMEM_SHARED((_N,), jnp.float32),
          pltpu.VMEM_SHARED((_N,), jnp.float32),
          pltpu.VMEM_SHARED((_T * 16,), jnp.float32),
          pltpu.VMEM((_N,), jnp.float32),
          pltpu.VMEM((_N,), jnp.float32),
          pltpu.VMEM((_N,), jnp.float32),
          pltpu.VMEM((_N,), jnp.float32),
          pltpu.SemaphoreType.DMA,
      ],
  )
  out = f(x, i_seq)
  return out[0].reshape(())
